# Initial kernel scaffold; baseline (speedup 1.0000x reference)
import functools

import jax
import jax.numpy as jnp
from jax import lax
from jax.experimental import pallas as pl
from jax.experimental.pallas import tpu as pltpu
from jax.experimental.pallas import tpu_sc as plsc

N_NODES = 10000
N_EDGES = 320000
D = 128
W_AUG = 144

NUM_CORES = 2
NUM_SUBCORES = 16
NUM_WORKERS = NUM_CORES * NUM_SUBCORES
E_PER_W = N_EDGES // NUM_WORKERS
CHUNK = 400
N_CHUNKS = E_PER_W // CHUNK
ROWS_PER_TILE = N_NODES // NUM_SUBCORES


def _make_sc_aggregate(width: int):
  mesh = plsc.VectorSubcoreMesh(core_axis_name="c", subcore_axis_name="s")

  @functools.partial(
      pl.kernel,
      out_type=jax.ShapeDtypeStruct((NUM_CORES, N_NODES, width), jnp.float32),
      mesh=mesh,
      scratch_types=[
          pltpu.VMEM((CHUNK,), jnp.int32),
          pltpu.VMEM((CHUNK,), jnp.int32),
          pltpu.VMEM((ROWS_PER_TILE, width), jnp.float32),
          pltpu.VMEM_SHARED((N_NODES, width), jnp.float32),
          pltpu.SemaphoreType.DMA,
      ],
  )
  def sc_agg(table_hbm, src_hbm, dst_hbm, zeros_hbm, out_hbm,
             src_v, dst_v, rows_v, acc_sh, sem):
    cid = lax.axis_index("c")
    sid = lax.axis_index("s")
    wid = sid * NUM_CORES + cid
    row0 = sid * ROWS_PER_TILE

    pltpu.sync_copy(zeros_hbm.at[pl.ds(row0, ROWS_PER_TILE)],
                    acc_sh.at[pl.ds(row0, ROWS_PER_TILE)])
    plsc.subcore_barrier()

    base = wid * E_PER_W

    def chunk_body(k, carry):
      off = base + k * CHUNK
      pltpu.sync_copy(src_hbm.at[pl.ds(off, CHUNK)], src_v)
      pltpu.sync_copy(dst_hbm.at[pl.ds(off, CHUNK)], dst_v)
      pltpu.async_copy(table_hbm.at[src_v], rows_v.at[pl.ds(0, CHUNK)],
                       sem).wait()
      pltpu.sync_copy(rows_v.at[pl.ds(0, CHUNK)], acc_sh.at[dst_v], add=True)
      return carry

    lax.fori_loop(0, N_CHUNKS, chunk_body, 0, unroll=False)
    plsc.subcore_barrier()

    pltpu.sync_copy(acc_sh.at[pl.ds(row0, ROWS_PER_TILE)], rows_v)
    pltpu.sync_copy(rows_v, out_hbm.at[cid, pl.ds(row0, ROWS_PER_TILE)])

  return sc_agg


_sc_agg_aug = _make_sc_aggregate(W_AUG)
_sc_agg_plain = _make_sc_aggregate(D)


ROW_BLK = 2000


def _tc_layer1_body(p0_ref, p1_ref, x_ref, ws_ref, wn_ref, b_ref,
                    h1_ref, deginv_ref):
  agg = p0_ref[...] + p1_ref[...]
  deg = agg[:, D:D + 1]
  scale = 1.0 / jnp.maximum(deg, 1.0)
  hn = agg[:, :D] * scale
  acc = jnp.dot(x_ref[...], ws_ref[...], preferred_element_type=jnp.float32)
  acc += jnp.dot(hn, wn_ref[...], preferred_element_type=jnp.float32)
  acc += b_ref[...]
  h1_ref[...] = jnp.maximum(acc, 0.0)
  deginv_ref[...] = scale


def _tc_layer2_body(p0_ref, p1_ref, h1_ref, deginv_ref, ws_ref, wn_ref, b_ref,
                    out_ref):
  agg = p0_ref[...] + p1_ref[...]
  hn = agg * deginv_ref[...]
  acc = jnp.dot(h1_ref[...], ws_ref[...], preferred_element_type=jnp.float32)
  acc += jnp.dot(hn, wn_ref[...], preferred_element_type=jnp.float32)
  acc += b_ref[...]
  out_ref[...] = acc


def _row_spec(width):
  return pl.BlockSpec((ROW_BLK, width), lambda i: (i, 0))


def _deginv_spec():
  return pl.BlockSpec((ROW_BLK, 1), lambda i: (i, 0))


def _full_spec(shape):
  return pl.BlockSpec(shape, lambda i: (0,) * len(shape))


def _tc_layer1(p0, p1, x, ws, wn, b):
  return pl.pallas_call(
      _tc_layer1_body,
      grid=(N_NODES // ROW_BLK,),
      in_specs=[
          _row_spec(W_AUG), _row_spec(W_AUG), _row_spec(D),
          _full_spec((D, D)), _full_spec((D, D)), _full_spec((1, D)),
      ],
      out_specs=[_row_spec(D), _deginv_spec()],
      out_shape=[
          jax.ShapeDtypeStruct((N_NODES, D), jnp.float32),
          jax.ShapeDtypeStruct((N_NODES, 1), jnp.float32),
      ],
  )(p0, p1, x, ws, wn, b)


def _tc_layer2(p0, p1, h1, deginv, ws, wn, b):
  return pl.pallas_call(
      _tc_layer2_body,
      grid=(N_NODES // ROW_BLK,),
      in_specs=[
          _row_spec(D), _row_spec(D), _row_spec(D), _deginv_spec(),
          _full_spec((D, D)), _full_spec((D, D)), _full_spec((1, D)),
      ],
      out_specs=_row_spec(D),
      out_shape=jax.ShapeDtypeStruct((N_NODES, D), jnp.float32),
  )(p0, p1, h1, deginv, ws, wn, b)


@jax.jit
def kernel(inputs, edge_index, W_self1, W_neigh1, b1, W_self2, W_neigh2, b2):
  src = edge_index[0]
  dst = edge_index[1]

  ones_col = jnp.ones((N_NODES, 1), jnp.float32)
  pad = jnp.zeros((N_NODES, W_AUG - D - 1), jnp.float32)
  xa = jnp.concatenate([inputs, ones_col, pad], axis=1)

  zeros_aug = jnp.zeros((N_NODES, W_AUG), jnp.float32)
  zeros_plain = jnp.zeros((N_NODES, D), jnp.float32)

  p = _sc_agg_aug(xa, src, dst, zeros_aug)
  h1, deginv = _tc_layer1(p[0], p[1], inputs,
                          W_self1, W_neigh1, b1.reshape(1, D))

  p2 = _sc_agg_plain(h1, src, dst, zeros_plain)
  out = _tc_layer2(p2[0], p2[1], h1, deginv,
                   W_self2, W_neigh2, b2.reshape(1, D))
  return out


# speedup vs baseline: 7.0621x; 7.0621x over previous
import functools

import jax
import jax.numpy as jnp
from jax import lax
from jax.experimental import pallas as pl
from jax.experimental.pallas import tpu as pltpu
from jax.experimental.pallas import tpu_sc as plsc

N_NODES = 10000
N_EDGES = 320000
D = 128

NUM_CORES = 2
NUM_SUBCORES = 16
E_PER_TILE = N_EDGES // NUM_SUBCORES
CHUNK = 400
N_CHUNKS = E_PER_TILE // CHUNK
ROWS_PER_TILE = N_NODES // NUM_SUBCORES

W1 = 80
W2 = 64


def _make_sc_aggregate(width: int):
  mesh = plsc.VectorSubcoreMesh(core_axis_name="c", subcore_axis_name="s",
                                num_cores=NUM_CORES, num_subcores=NUM_SUBCORES)

  @functools.partial(
      pl.kernel,
      out_type=jax.ShapeDtypeStruct((NUM_CORES, N_NODES, width), jnp.float32),
      mesh=mesh,
      scratch_types=[
          pltpu.VMEM((CHUNK,), jnp.int32),
          pltpu.VMEM((CHUNK,), jnp.int32),
          pltpu.VMEM((ROWS_PER_TILE, width), jnp.float32),
          pltpu.VMEM_SHARED((N_NODES, width), jnp.float32),
          pltpu.SemaphoreType.DMA,
      ],
      compiler_params=pltpu.CompilerParams(use_tc_tiling_on_sc=False),
  )
  def sc_agg(t0_hbm, t1_hbm, src_hbm, dst_hbm, zeros_hbm, out_hbm,
             src_v, dst_v, rows_v, acc_sh, sem):
    cid = lax.axis_index("c")
    sid = lax.axis_index("s")
    row0 = sid * ROWS_PER_TILE

    pltpu.sync_copy(zeros_hbm.at[pl.ds(row0, ROWS_PER_TILE)],
                    acc_sh.at[pl.ds(row0, ROWS_PER_TILE)])
    plsc.subcore_barrier()

    base = sid * E_PER_TILE

    def make_body(table_hbm):
      def chunk_body(k, carry):
        off = base + k * CHUNK
        pltpu.sync_copy(src_hbm.at[pl.ds(off, CHUNK)], src_v)
        pltpu.sync_copy(dst_hbm.at[pl.ds(off, CHUNK)], dst_v)
        pltpu.async_copy(table_hbm.at[src_v], rows_v.at[pl.ds(0, CHUNK)],
                         sem).wait()
        pltpu.sync_copy(rows_v.at[pl.ds(0, CHUNK)], acc_sh.at[dst_v], add=True)
        return carry
      return chunk_body

    @pl.when(cid == 0)
    def _():
      lax.fori_loop(0, N_CHUNKS, make_body(t0_hbm), 0, unroll=False)

    @pl.when(cid == 1)
    def _():
      lax.fori_loop(0, N_CHUNKS, make_body(t1_hbm), 0, unroll=False)

    plsc.subcore_barrier()

    pltpu.sync_copy(acc_sh.at[pl.ds(row0, ROWS_PER_TILE)], rows_v)
    pltpu.sync_copy(rows_v, out_hbm.at[cid, pl.ds(row0, ROWS_PER_TILE)])

  return sc_agg


_sc_agg_cache = {}


def _sc_aggregate(width: int):
  if width not in _sc_agg_cache:
    _sc_agg_cache[width] = _make_sc_aggregate(width)
  return _sc_agg_cache[width]


ROW_BLK = 2000


def _tc_layer1_body(p0_ref, p1_ref, x_ref, ws_ref, wn_ref, b_ref,
                    h1_ref, deginv_ref):
  deg = p1_ref[:, 64:65]
  scale = 1.0 / jnp.maximum(deg, 1.0)
  hn = jnp.concatenate([p0_ref[:, 0:64], p1_ref[:, 0:64]], axis=1) * scale
  acc = jnp.dot(x_ref[...], ws_ref[...], preferred_element_type=jnp.float32)
  acc += jnp.dot(hn, wn_ref[...], preferred_element_type=jnp.float32)
  acc += b_ref[...]
  h1_ref[...] = jnp.maximum(acc, 0.0)
  deginv_ref[...] = scale


def _tc_layer2_body(p0_ref, p1_ref, h1_ref, deginv_ref, ws_ref, wn_ref, b_ref,
                    out_ref):
  hn = jnp.concatenate([p0_ref[...], p1_ref[...]], axis=1) * deginv_ref[...]
  acc = jnp.dot(h1_ref[...], ws_ref[...], preferred_element_type=jnp.float32)
  acc += jnp.dot(hn, wn_ref[...], preferred_element_type=jnp.float32)
  acc += b_ref[...]
  out_ref[...] = acc


def _row_spec(width):
  return pl.BlockSpec((ROW_BLK, width), lambda i: (i, 0))


def _full_spec(shape):
  return pl.BlockSpec(shape, lambda i: (0,) * len(shape))


def _tc_layer1(p0, p1, x, ws, wn, b):
  return pl.pallas_call(
      _tc_layer1_body,
      grid=(N_NODES // ROW_BLK,),
      in_specs=[
          _row_spec(W1), _row_spec(W1), _row_spec(D),
          _full_spec((D, D)), _full_spec((D, D)), _full_spec((1, D)),
      ],
      out_specs=[_row_spec(D), _row_spec(1)],
      out_shape=[
          jax.ShapeDtypeStruct((N_NODES, D), jnp.float32),
          jax.ShapeDtypeStruct((N_NODES, 1), jnp.float32),
      ],
  )(p0, p1, x, ws, wn, b)


def _tc_layer2(p0, p1, h1, deginv, ws, wn, b):
  return pl.pallas_call(
      _tc_layer2_body,
      grid=(N_NODES // ROW_BLK,),
      in_specs=[
          _row_spec(W2), _row_spec(W2), _row_spec(D), _row_spec(1),
          _full_spec((D, D)), _full_spec((D, D)), _full_spec((1, D)),
      ],
      out_specs=_row_spec(D),
      out_shape=jax.ShapeDtypeStruct((N_NODES, D), jnp.float32),
  )(p0, p1, h1, deginv, ws, wn, b)


@jax.jit
def kernel(inputs, edge_index, W_self1, W_neigh1, b1, W_self2, W_neigh2, b2):
  src = edge_index[0]
  dst = edge_index[1]

  ones_col = jnp.ones((N_NODES, 1), jnp.float32)
  pad = jnp.zeros((N_NODES, W1 - 64 - 1), jnp.float32)
  t0 = inputs[:, 0:W1]
  t1 = jnp.concatenate([inputs[:, 64:D], ones_col, pad], axis=1)

  zeros1 = jnp.zeros((N_NODES, W1), jnp.float32)
  zeros2 = jnp.zeros((N_NODES, W2), jnp.float32)

  p = _sc_aggregate(W1)(t0, t1, src, dst, zeros1)
  h1, deginv = _tc_layer1(p[0], p[1], inputs,
                          W_self1, W_neigh1, b1.reshape(1, D))

  p2 = _sc_aggregate(W2)(h1[:, 0:W2], h1[:, W2:D], src, dst, zeros2)
  out = _tc_layer2(p2[0], p2[1], h1, deginv,
                   W_self2, W_neigh2, b2.reshape(1, D))
  return out


# speedup vs baseline: 10.2818x; 1.4559x over previous
import functools

import jax
import jax.numpy as jnp
from jax import lax
from jax.experimental import pallas as pl
from jax.experimental.pallas import tpu as pltpu
from jax.experimental.pallas import tpu_sc as plsc

N_NODES = 10000
N_EDGES = 320000
D = 128

NUM_CORES = 2
NUM_SUBCORES = 16
E_PER_TILE = N_EDGES // NUM_SUBCORES
ROWS_PER_TILE = N_NODES // NUM_SUBCORES

W1 = 80
W2 = 64
CHUNK1 = 200
CHUNK2 = 400


def _make_sc_aggregate(width: int, chunk: int):
  mesh = plsc.VectorSubcoreMesh(core_axis_name="c", subcore_axis_name="s",
                                num_cores=NUM_CORES, num_subcores=NUM_SUBCORES)

  n_chunks = E_PER_TILE // chunk

  @functools.partial(
      pl.kernel,
      out_type=jax.ShapeDtypeStruct((NUM_CORES, N_NODES, width), jnp.float32),
      mesh=mesh,
      scratch_types=[
          pltpu.VMEM((chunk,), jnp.int32),
          pltpu.VMEM((chunk,), jnp.int32),
          pltpu.VMEM((chunk,), jnp.int32),
          pltpu.VMEM((chunk,), jnp.int32),
          pltpu.VMEM((chunk, width), jnp.float32),
          pltpu.VMEM((chunk, width), jnp.float32),
          pltpu.VMEM_SHARED((N_NODES, width), jnp.float32),
          pltpu.SemaphoreType.DMA,
          pltpu.SemaphoreType.DMA,
          pltpu.SemaphoreType.DMA,
          pltpu.SemaphoreType.DMA,
      ],
      compiler_params=pltpu.CompilerParams(use_tc_tiling_on_sc=False),
  )
  def sc_agg(t0_hbm, t1_hbm, src_hbm, dst_hbm, zeros_hbm, out_hbm,
             src0, src1, dst0, dst1, buf0, buf1, acc_sh,
             isem0, isem1, rsem0, rsem1):
    cid = lax.axis_index("c")
    sid = lax.axis_index("s")
    row0 = sid * ROWS_PER_TILE
    base = sid * E_PER_TILE

    ibufs = ((src0, dst0), (src1, dst1))
    isems = (isem0, isem1)
    rbufs = (buf0, buf1)
    rsems = (rsem0, rsem1)

    pltpu.sync_copy(zeros_hbm.at[pl.ds(row0, ROWS_PER_TILE)],
                    acc_sh.at[pl.ds(row0, ROWS_PER_TILE)])
    plsc.subcore_barrier()

    def idx_load(k, b):
      off = base + k * chunk
      return (pltpu.make_async_copy(src_hbm.at[pl.ds(off, chunk)],
                                    ibufs[b][0], isems[b]),
              pltpu.make_async_copy(dst_hbm.at[pl.ds(off, chunk)],
                                    ibufs[b][1], isems[b]))

    def run_edges(table_hbm):
      def gather(b):
        return pltpu.make_async_copy(table_hbm.at[ibufs[b][0]], rbufs[b],
                                     rsems[b])

      ia, ib = idx_load(0, 0)
      ia.start(); ib.start(); ia.wait(); ib.wait()
      ia, ib = idx_load(1, 1)
      ia.start(); ib.start()
      gather(0).start()

      def body(j, carry):
        for b in range(2):
          k = 2 * j + b

          @pl.when(k + 1 < n_chunks)
          def _():
            ia, ib = idx_load(k + 1, 1 - b)
            ia.wait(); ib.wait()
            gather(1 - b).start()

          gather(b).wait()
          pltpu.sync_copy(rbufs[b], acc_sh.at[ibufs[b][1]], add=True)

          @pl.when(k + 2 < n_chunks)
          def _():
            ia, ib = idx_load(k + 2, b)
            ia.start(); ib.start()
        return carry

      lax.fori_loop(0, n_chunks // 2, body, 0, unroll=False)

    @pl.when(cid == 0)
    def _():
      run_edges(t0_hbm)

    @pl.when(cid == 1)
    def _():
      run_edges(t1_hbm)

    plsc.subcore_barrier()

    pltpu.sync_copy(acc_sh.at[pl.ds(row0, ROWS_PER_TILE)],
                    out_hbm.at[cid, pl.ds(row0, ROWS_PER_TILE)])

  return sc_agg


_sc_agg_cache = {}


def _sc_aggregate(width: int, chunk: int):
  if (width, chunk) not in _sc_agg_cache:
    _sc_agg_cache[(width, chunk)] = _make_sc_aggregate(width, chunk)
  return _sc_agg_cache[(width, chunk)]


ROW_BLK = 2000


def _tc_layer1_body(p0_ref, p1_ref, x_ref, ws_ref, wn_ref, b_ref,
                    h1_ref, deginv_ref):
  deg = p1_ref[:, 64:65]
  scale = 1.0 / jnp.maximum(deg, 1.0)
  hn = jnp.concatenate([p0_ref[:, 0:64], p1_ref[:, 0:64]], axis=1) * scale
  acc = jnp.dot(x_ref[...], ws_ref[...], preferred_element_type=jnp.float32)
  acc += jnp.dot(hn, wn_ref[...], preferred_element_type=jnp.float32)
  acc += b_ref[...]
  h1_ref[...] = jnp.maximum(acc, 0.0)
  deginv_ref[...] = scale


def _tc_layer2_body(p0_ref, p1_ref, h1_ref, deginv_ref, ws_ref, wn_ref, b_ref,
                    out_ref):
  hn = jnp.concatenate([p0_ref[...], p1_ref[...]], axis=1) * deginv_ref[...]
  acc = jnp.dot(h1_ref[...], ws_ref[...], preferred_element_type=jnp.float32)
  acc += jnp.dot(hn, wn_ref[...], preferred_element_type=jnp.float32)
  acc += b_ref[...]
  out_ref[...] = acc


def _row_spec(width):
  return pl.BlockSpec((ROW_BLK, width), lambda i: (i, 0))


def _full_spec(shape):
  return pl.BlockSpec(shape, lambda i: (0,) * len(shape))


def _tc_layer1(p0, p1, x, ws, wn, b):
  return pl.pallas_call(
      _tc_layer1_body,
      grid=(N_NODES // ROW_BLK,),
      in_specs=[
          _row_spec(W1), _row_spec(W1), _row_spec(D),
          _full_spec((D, D)), _full_spec((D, D)), _full_spec((1, D)),
      ],
      out_specs=[_row_spec(D), _row_spec(1)],
      out_shape=[
          jax.ShapeDtypeStruct((N_NODES, D), jnp.float32),
          jax.ShapeDtypeStruct((N_NODES, 1), jnp.float32),
      ],
  )(p0, p1, x, ws, wn, b)


def _tc_layer2(p0, p1, h1, deginv, ws, wn, b):
  return pl.pallas_call(
      _tc_layer2_body,
      grid=(N_NODES // ROW_BLK,),
      in_specs=[
          _row_spec(W2), _row_spec(W2), _row_spec(D), _row_spec(1),
          _full_spec((D, D)), _full_spec((D, D)), _full_spec((1, D)),
      ],
      out_specs=_row_spec(D),
      out_shape=jax.ShapeDtypeStruct((N_NODES, D), jnp.float32),
  )(p0, p1, h1, deginv, ws, wn, b)


@jax.jit
def kernel(inputs, edge_index, W_self1, W_neigh1, b1, W_self2, W_neigh2, b2):
  src = edge_index[0]
  dst = edge_index[1]

  ones_col = jnp.ones((N_NODES, 1), jnp.float32)
  pad = jnp.zeros((N_NODES, W1 - 64 - 1), jnp.float32)
  t0 = inputs[:, 0:W1]
  t1 = jnp.concatenate([inputs[:, 64:D], ones_col, pad], axis=1)

  zeros1 = jnp.zeros((N_NODES, W1), jnp.float32)
  zeros2 = jnp.zeros((N_NODES, W2), jnp.float32)

  p = _sc_aggregate(W1, CHUNK1)(t0, t1, src, dst, zeros1)
  h1, deginv = _tc_layer1(p[0], p[1], inputs,
                          W_self1, W_neigh1, b1.reshape(1, D))

  p2 = _sc_aggregate(W2, CHUNK2)(h1[:, 0:W2], h1[:, W2:D], src, dst, zeros2)
  out = _tc_layer2(p2[0], p2[1], h1, deginv,
                   W_self2, W_neigh2, b2.reshape(1, D))
  return out


# speedup vs baseline: 11.8359x; 1.1511x over previous
import functools

import jax
import jax.numpy as jnp
from jax import lax
from jax.experimental import pallas as pl
from jax.experimental.pallas import tpu as pltpu
from jax.experimental.pallas import tpu_sc as plsc

N_NODES = 10000
N_EDGES = 320000
D = 128

NUM_CORES = 2
NUM_SUBCORES = 16
E_PER_TILE = N_EDGES // NUM_SUBCORES
ROWS_PER_TILE = N_NODES // NUM_SUBCORES

W1 = 80
W2 = 64
CHUNK1 = 200
CHUNK2 = 400


def _make_sc_aggregate(width: int, chunk: int, with_deg: bool):
  mesh = plsc.VectorSubcoreMesh(core_axis_name="c", subcore_axis_name="s",
                                num_cores=NUM_CORES, num_subcores=NUM_SUBCORES)
  n_chunks = E_PER_TILE // chunk

  out_type = [jax.ShapeDtypeStruct((N_NODES, D), jnp.float32)]
  if with_deg:
    out_type.append(jax.ShapeDtypeStruct((N_NODES, D), jnp.float32))

  @functools.partial(
      pl.kernel,
      out_type=tuple(out_type),
      mesh=mesh,
      scratch_types=[
          pltpu.VMEM((chunk,), jnp.int32),
          pltpu.VMEM((chunk,), jnp.int32),
          pltpu.VMEM((chunk,), jnp.int32),
          pltpu.VMEM((chunk,), jnp.int32),
          pltpu.VMEM((chunk, width), jnp.float32),
          pltpu.VMEM((chunk, width), jnp.float32),
          pltpu.VMEM_SHARED((N_NODES, width), jnp.float32),
          pltpu.SemaphoreType.DMA,
          pltpu.SemaphoreType.DMA,
          pltpu.SemaphoreType.DMA,
          pltpu.SemaphoreType.DMA,
      ],
      compiler_params=pltpu.CompilerParams(use_tc_tiling_on_sc=False),
  )
  def sc_agg(t0_hbm, t1_hbm, eidx_hbm, zeros_hbm, *outs_and_scratch):
    if with_deg:
      out_hbm, deg_hbm = outs_and_scratch[0], outs_and_scratch[1]
      scratch = outs_and_scratch[2:]
    else:
      out_hbm = outs_and_scratch[0]
      scratch = outs_and_scratch[1:]
    (src0, src1, dst0, dst1, buf0, buf1, acc_sh,
     isem0, isem1, rsem0, rsem1) = scratch

    cid = lax.axis_index("c")
    sid = lax.axis_index("s")
    row0 = sid * ROWS_PER_TILE
    base = sid * E_PER_TILE

    ibufs = ((src0, dst0), (src1, dst1))
    isems = (isem0, isem1)
    rbufs = (buf0, buf1)
    rsems = (rsem0, rsem1)

    pltpu.sync_copy(zeros_hbm.at[pl.ds(row0, ROWS_PER_TILE)],
                    acc_sh.at[pl.ds(row0, ROWS_PER_TILE)])
    plsc.subcore_barrier()

    def idx_load(k, b):
      off = base + k * chunk
      return (pltpu.make_async_copy(eidx_hbm.at[0, pl.ds(off, chunk)],
                                    ibufs[b][0], isems[b]),
              pltpu.make_async_copy(eidx_hbm.at[1, pl.ds(off, chunk)],
                                    ibufs[b][1], isems[b]))

    def run_edges(table_hbm):
      def gather(b):
        return pltpu.make_async_copy(table_hbm.at[ibufs[b][0]], rbufs[b],
                                     rsems[b])

      ia, ib = idx_load(0, 0)
      ia.start(); ib.start(); ia.wait(); ib.wait()
      ia, ib = idx_load(1, 1)
      ia.start(); ib.start()
      gather(0).start()

      def body(j, carry):
        for b in range(2):
          k = 2 * j + b

          @pl.when(k + 1 < n_chunks)
          def _():
            ia, ib = idx_load(k + 1, 1 - b)
            ia.wait(); ib.wait()
            gather(1 - b).start()

          gather(b).wait()
          pltpu.sync_copy(rbufs[b], acc_sh.at[ibufs[b][1]], add=True)

          @pl.when(k + 2 < n_chunks)
          def _():
            ia, ib = idx_load(k + 2, b)
            ia.start(); ib.start()
        return carry

      lax.fori_loop(0, n_chunks // 2, body, 0, unroll=False)

    @pl.when(cid == 0)
    def _():
      run_edges(t0_hbm)

    @pl.when(cid == 1)
    def _():
      run_edges(t1_hbm)

    plsc.subcore_barrier()

    rows = pl.ds(row0, ROWS_PER_TILE)

    @pl.when(cid == 0)
    def _():
      pltpu.sync_copy(acc_sh.at[rows, pl.ds(0, 64)],
                      out_hbm.at[rows, pl.ds(0, 64)])

    @pl.when(cid == 1)
    def _():
      pltpu.sync_copy(acc_sh.at[rows, pl.ds(0, 64)],
                      out_hbm.at[rows, pl.ds(64, 64)])
      if with_deg:
        pltpu.sync_copy(acc_sh.at[rows, pl.ds(64, 16)],
                        deg_hbm.at[rows, pl.ds(0, 16)])

  return sc_agg


_sc_agg_cache = {}


def _sc_aggregate(width: int, chunk: int, with_deg: bool):
  key = (width, chunk, with_deg)
  if key not in _sc_agg_cache:
    _sc_agg_cache[key] = _make_sc_aggregate(width, chunk, with_deg)
  return _sc_agg_cache[key]


ROW_BLK = 2000


def _tc_layer1_body(agg_ref, deg_ref, x_ref, ws_ref, wn_ref, b_ref,
                    h1a_ref, h1b_ref, deginv_ref):
  deg = deg_ref[:, 0:1]
  scale = 1.0 / jnp.maximum(deg, 1.0)
  hn = agg_ref[...] * scale
  acc = jnp.dot(x_ref[...], ws_ref[...], preferred_element_type=jnp.float32)
  acc += jnp.dot(hn, wn_ref[...], preferred_element_type=jnp.float32)
  acc += b_ref[...]
  h1 = jnp.maximum(acc, 0.0)
  h1a_ref[...] = h1[:, 0:64]
  h1b_ref[...] = h1[:, 64:128]
  deginv_ref[...] = scale


def _tc_layer2_body(agg_ref, h1a_ref, h1b_ref, deginv_ref, ws_ref, wn_ref,
                    b_ref, out_ref):
  h1 = jnp.concatenate([h1a_ref[...], h1b_ref[...]], axis=1)
  hn = agg_ref[...] * deginv_ref[...]
  acc = jnp.dot(h1, ws_ref[...], preferred_element_type=jnp.float32)
  acc += jnp.dot(hn, wn_ref[...], preferred_element_type=jnp.float32)
  acc += b_ref[...]
  out_ref[...] = acc


def _row_spec(width):
  return pl.BlockSpec((ROW_BLK, width), lambda i: (i, 0))


def _full_spec(shape):
  return pl.BlockSpec(shape, lambda i: (0,) * len(shape))


def _tc_layer1(agg, degblk, x, ws, wn, b):
  return pl.pallas_call(
      _tc_layer1_body,
      grid=(N_NODES // ROW_BLK,),
      in_specs=[
          _row_spec(D), _row_spec(D), _row_spec(D),
          _full_spec((D, D)), _full_spec((D, D)), _full_spec((1, D)),
      ],
      out_specs=[_row_spec(64), _row_spec(64), _row_spec(1)],
      out_shape=[
          jax.ShapeDtypeStruct((N_NODES, 64), jnp.float32),
          jax.ShapeDtypeStruct((N_NODES, 64), jnp.float32),
          jax.ShapeDtypeStruct((N_NODES, 1), jnp.float32),
      ],
  )(agg, degblk, x, ws, wn, b)


def _tc_layer2(agg, h1a, h1b, deginv, ws, wn, b):
  return pl.pallas_call(
      _tc_layer2_body,
      grid=(N_NODES // ROW_BLK,),
      in_specs=[
          _row_spec(D), _row_spec(64), _row_spec(64), _row_spec(1),
          _full_spec((D, D)), _full_spec((D, D)), _full_spec((1, D)),
      ],
      out_specs=_row_spec(D),
      out_shape=jax.ShapeDtypeStruct((N_NODES, D), jnp.float32),
  )(agg, h1a, h1b, deginv, ws, wn, b)


@jax.jit
def kernel(inputs, edge_index, W_self1, W_neigh1, b1, W_self2, W_neigh2, b2):
  ones_col = jnp.ones((N_NODES, 1), jnp.float32)
  pad = jnp.zeros((N_NODES, W1 - 64 - 1), jnp.float32)
  t0 = inputs[:, 0:W1]
  t1 = jnp.concatenate([inputs[:, 64:D], ones_col, pad], axis=1)

  zeros1 = jnp.zeros((N_NODES, W1), jnp.float32)
  zeros2 = jnp.zeros((N_NODES, W2), jnp.float32)

  agg1, degblk = _sc_aggregate(W1, CHUNK1, True)(t0, t1, edge_index, zeros1)
  h1a, h1b, deginv = _tc_layer1(agg1, degblk, inputs,
                                W_self1, W_neigh1, b1.reshape(1, D))

  agg2, = _sc_aggregate(W2, CHUNK2, False)(h1a, h1b, edge_index, zeros2)
  out = _tc_layer2(agg2, h1a, h1b, deginv,
                   W_self2, W_neigh2, b2.reshape(1, D))
  return out
